# Initial kernel scaffold; baseline (speedup 1.0000x reference)
import jax
import jax.numpy as jnp
from jax.experimental import pallas as pl


def kernel(hidden_states, W_g, W1, b1, W2, b2):
    raise NotImplementedError("write your pallas kernel here")


# speedup vs baseline: 1.5657x; 1.5657x over previous
import functools

import jax
import jax.numpy as jnp
from jax.experimental import pallas as pl
from jax.experimental.pallas import tpu as pltpu

B, S, H = 2, 2048, 1024
E, K, F = 8, 2, 1024
T = B * S
BS = 512
NT = T // BS


def _moe_block(x_ref, wg_ref, w1_ref, b1_ref, w2_ref, b2_ref, out_ref):
    e = pl.program_id(1)

    @pl.when(e == 0)
    def _init():
        out_ref[...] = jnp.zeros_like(out_ref)

    xb = x_ref[...]
    logits = jnp.dot(xb, wg_ref[...], preferred_element_type=jnp.float32)
    probs = jax.nn.softmax(logits, axis=-1)

    idx = jax.lax.broadcasted_iota(jnp.int32, (BS, E), 1)
    m0 = jnp.max(probs, axis=1, keepdims=True)
    i0 = jnp.min(jnp.where(probs == m0, idx, E), axis=1, keepdims=True)
    probs2 = jnp.where(idx == i0, -jnp.inf, probs)
    m1 = jnp.max(probs2, axis=1, keepdims=True)
    i1 = jnp.min(jnp.where(probs2 == m1, idx, E), axis=1, keepdims=True)
    wsum = m0 + m1
    w0 = m0 / wsum
    w1 = m1 / wsum
    ce = jnp.where(i0 == e, w0, 0.0) + jnp.where(i1 == e, w1, 0.0)

    h = jnp.maximum(
        jnp.dot(xb, w1_ref[0], preferred_element_type=jnp.float32) + b1_ref[0],
        0.0,
    )
    y = jnp.dot(h, w2_ref[0], preferred_element_type=jnp.float32) + b2_ref[0]
    out_ref[...] += ce * y


@jax.jit
def _moe(x, W_g, W1, b1, W2, b2):
    return pl.pallas_call(
        _moe_block,
        grid=(NT, E),
        in_specs=[
            pl.BlockSpec((BS, H), lambda t, e: (t, 0)),
            pl.BlockSpec((H, E), lambda t, e: (0, 0)),
            pl.BlockSpec((1, H, F), lambda t, e: (e, 0, 0)),
            pl.BlockSpec((1, 1, F), lambda t, e: (e, 0, 0)),
            pl.BlockSpec((1, F, H), lambda t, e: (e, 0, 0)),
            pl.BlockSpec((1, 1, H), lambda t, e: (e, 0, 0)),
        ],
        out_specs=pl.BlockSpec((BS, H), lambda t, e: (t, 0)),
        out_shape=jax.ShapeDtypeStruct((T, H), jnp.float32),
    )(x, W_g, W1, b1[:, None, :], W2, b2[:, None, :])


def kernel(hidden_states, W_g, W1, b1, W2, b2):
    orig_shape = hidden_states.shape
    x = hidden_states.reshape(-1, orig_shape[-1])
    out = _moe(x, W_g, W1, b1, W2, b2)
    return out.reshape(orig_shape)
